# baseline (speedup 1.0000x reference)
import functools

import jax
import jax.numpy as jnp
from jax import lax
from jax.experimental import pallas as pl
from jax.experimental.pallas import tpu as pltpu
from jax.experimental.pallas import tpu_sc as plsc

B, T, N, F = 16, 20, 50000, 2
H = 10
R = 64
RP = 128
NB = 4096
NSTEPS = (N + NB - 1) // NB
NPAD = NSTEPS * NB

_SC = plsc.get_sparse_core_info()
NC, NS, LANES = _SC.num_cores, _SC.num_subcores, _SC.num_lanes
NW = NC * NS
CH = NPAD // NW


def _tc_kernel(x_ref, out_pred_ref, out_flat_ref):
    i = pl.program_id(0)

    wr = jax.lax.broadcasted_iota(jnp.int32, (B, B * T), 0)
    wc = jax.lax.broadcasted_iota(jnp.int32, (B, B * T), 1)
    w = jnp.where(wc // T == wr, 1.0 / T, 0.0)
    x = x_ref[...].reshape(B * T, NB)
    pred = jax.lax.dot_general(
        w, x, (((1,), (0,)), ((), ())),
        preferred_element_type=jnp.float32)

    node_idx = i * NB + jax.lax.broadcasted_iota(jnp.int32, (1, NB), 1)
    pred = jnp.where(node_idx < N, pred, 0.0)

    out_pred_ref[...] = jnp.broadcast_to(pred[None, :, :], (H, B, NB))
    out_flat_ref[...] = pred


_sc_mesh = plsc.VectorSubcoreMesh(core_axis_name="c", subcore_axis_name="s")


@functools.partial(
    pl.kernel,
    mesh=_sc_mesh,
    compiler_params=pltpu.CompilerParams(needs_layout_passes=False),
    out_type=[
        jax.ShapeDtypeStruct((NW, B + 1, R), jnp.float32),
    ],
    scratch_types=[
        pltpu.VMEM((CH,), jnp.int32),
        pltpu.VMEM((CH,), jnp.float32),
        pltpu.VMEM((RP,), jnp.float32),
        pltpu.VMEM((B + 1, R), jnp.float32),
        pltpu.VMEM((RP,), jnp.float32),
    ],
)
def _sc_segment(pred_hbm, cid_hbm, sums_hbm,
                cid_v, val_v, bins_v, sums_v, cnt_v):
    c = lax.axis_index("c")
    s = lax.axis_index("s")
    wid = s * NC + c
    base = wid * CH
    zeros16 = jnp.zeros((LANES,), jnp.float32)
    ones16 = jnp.ones((LANES,), jnp.float32)

    pltpu.sync_copy(cid_hbm.at[pl.ds(base, CH)], cid_v)

    for k in range(RP // LANES):
        cnt_v[pl.ds(k * LANES, LANES)] = zeros16

    def _cnt_body(j, carry):
        idx = cid_v[pl.ds(j * LANES, LANES)]
        plsc.addupdate_scatter(cnt_v, [idx], ones16)
        return carry

    lax.fori_loop(0, CH // LANES, _cnt_body, 0)

    for b in range(B):
        pltpu.sync_copy(pred_hbm.at[b, pl.ds(base, CH)], val_v)
        for k in range(RP // LANES):
            bins_v[pl.ds(k * LANES, LANES)] = zeros16

        def _sum_body(j, carry):
            idx = cid_v[pl.ds(j * LANES, LANES)]
            vals = val_v[pl.ds(j * LANES, LANES)]
            plsc.addupdate_scatter(bins_v, [idx], vals)
            return carry

        lax.fori_loop(0, CH // LANES, _sum_body, 0)
        for k in range(R // LANES):
            sums_v[b, pl.ds(k * LANES, LANES)] = bins_v[pl.ds(k * LANES, LANES)]

    for k in range(R // LANES):
        sums_v[B, pl.ds(k * LANES, LANES)] = cnt_v[pl.ds(k * LANES, LANES)]

    pltpu.sync_copy(sums_v, sums_hbm.at[wid])


def kernel(speed_seq, cluster_id):
    x0 = speed_seq[:, :, :, 0]
    cid = cluster_id.astype(jnp.int32)
    cid_pad = jnp.concatenate([cid, jnp.full((NPAD - N,), R, dtype=jnp.int32)])

    pred_speed, pred_flat = pl.pallas_call(
        _tc_kernel,
        grid=(NSTEPS,),
        in_specs=[pl.BlockSpec((B, T, NB), lambda i: (0, 0, i))],
        out_specs=[
            pl.BlockSpec((H, B, NB), lambda i: (0, 0, i)),
            pl.BlockSpec((B, NB), lambda i: (0, i)),
        ],
        out_shape=[
            jax.ShapeDtypeStruct((H, B, N), jnp.float32),
            jax.ShapeDtypeStruct((B, NPAD), jnp.float32),
        ],
    )(x0)

    (partials,) = _sc_segment(pred_flat, cid_pad)
    combined = partials.sum(axis=0)
    regional = combined[:B] / combined[B][None, :]
    pred_speed_regional = jnp.broadcast_to(regional[:, None, :], (B, H, R))
    return pred_speed.transpose(1, 0, 2), pred_speed_regional


# speedup vs baseline: 1.0003x; 1.0003x over previous
import functools

import jax
import jax.numpy as jnp
from jax import lax
from jax.experimental import pallas as pl
from jax.experimental.pallas import tpu as pltpu
from jax.experimental.pallas import tpu_sc as plsc

B, T, N, F = 16, 20, 50000, 2
H = 10
R = 64
RP = 128
NB = 4096
NSTEPS = (N + NB - 1) // NB
NPAD = NSTEPS * NB

_SC = plsc.get_sparse_core_info()
NC, NS, LANES = _SC.num_cores, _SC.num_subcores, _SC.num_lanes
NW = NC * NS
CH = NPAD // NW
assert CH * NW == NPAD and CH % 8 == 0 and NPAD % NB == 0


def _tc_kernel(x_ref, out_pred_ref, out_flat_ref):
    i = pl.program_id(0)

    wr = jax.lax.broadcasted_iota(jnp.int32, (B, B * T), 0)
    wc = jax.lax.broadcasted_iota(jnp.int32, (B, B * T), 1)
    w = jnp.where(wc // T == wr, 1.0 / T, 0.0)
    x = x_ref[...].reshape(B * T, NB)
    pred = jax.lax.dot_general(
        w, x, (((1,), (0,)), ((), ())),
        preferred_element_type=jnp.float32)

    node_idx = i * NB + jax.lax.broadcasted_iota(jnp.int32, (1, NB), 1)
    pred = jnp.where(node_idx < N, pred, 0.0)

    out_pred_ref[...] = jnp.broadcast_to(pred[None, :, :], (H, B, NB))
    out_flat_ref[...] = pred


_sc_mesh = plsc.VectorSubcoreMesh(core_axis_name="c", subcore_axis_name="s")


@functools.partial(
    pl.kernel,
    mesh=_sc_mesh,
    compiler_params=pltpu.CompilerParams(needs_layout_passes=False),
    out_type=[
        jax.ShapeDtypeStruct((NW, B + 1, R), jnp.float32),
    ],
    scratch_types=[
        pltpu.VMEM((CH,), jnp.int32),
        pltpu.VMEM((CH,), jnp.float32),
        pltpu.VMEM((RP,), jnp.float32),
        pltpu.VMEM((B + 1, R), jnp.float32),
        pltpu.VMEM((RP,), jnp.float32),
    ],
)
def _sc_segment(pred_hbm, cid_hbm, sums_hbm,
                cid_v, val_v, bins_v, sums_v, cnt_v):
    c = lax.axis_index("c")
    s = lax.axis_index("s")
    wid = s * NC + c
    base = wid * CH
    zeros16 = jnp.zeros((LANES,), jnp.float32)
    ones16 = jnp.ones((LANES,), jnp.float32)

    pltpu.sync_copy(cid_hbm.at[pl.ds(base, CH)], cid_v)

    for k in range(RP // LANES):
        cnt_v[pl.ds(k * LANES, LANES)] = zeros16

    def _cnt_body(j, carry):
        idx = cid_v[pl.ds(j * LANES, LANES)]
        plsc.addupdate_scatter(cnt_v, [idx], ones16)
        return carry

    lax.fori_loop(0, CH // LANES, _cnt_body, 0)

    for b in range(B):
        pltpu.sync_copy(pred_hbm.at[b, pl.ds(base, CH)], val_v)
        for k in range(RP // LANES):
            bins_v[pl.ds(k * LANES, LANES)] = zeros16

        def _sum_body(j, carry):
            idx = cid_v[pl.ds(j * LANES, LANES)]
            vals = val_v[pl.ds(j * LANES, LANES)]
            plsc.addupdate_scatter(bins_v, [idx], vals)
            return carry

        lax.fori_loop(0, CH // LANES, _sum_body, 0)
        for k in range(R // LANES):
            sums_v[b, pl.ds(k * LANES, LANES)] = bins_v[pl.ds(k * LANES, LANES)]

    for k in range(R // LANES):
        sums_v[B, pl.ds(k * LANES, LANES)] = cnt_v[pl.ds(k * LANES, LANES)]

    pltpu.sync_copy(sums_v, sums_hbm.at[wid])


def kernel(speed_seq, cluster_id):
    x0 = speed_seq[:, :, :, 0]
    cid = cluster_id.astype(jnp.int32)
    cid_pad = jnp.concatenate([cid, jnp.full((NPAD - N,), R, dtype=jnp.int32)])

    pred_speed, pred_flat = pl.pallas_call(
        _tc_kernel,
        grid=(NSTEPS,),
        in_specs=[pl.BlockSpec((B, T, NB), lambda i: (0, 0, i))],
        out_specs=[
            pl.BlockSpec((H, B, NB), lambda i: (0, 0, i)),
            pl.BlockSpec((B, NB), lambda i: (0, i)),
        ],
        out_shape=[
            jax.ShapeDtypeStruct((H, B, N), jnp.float32),
            jax.ShapeDtypeStruct((B, NPAD), jnp.float32),
        ],
    )(x0)

    (partials,) = _sc_segment(pred_flat, cid_pad)
    combined = partials.sum(axis=0)
    regional = combined[:B] / combined[B][None, :]
    pred_speed_regional = jnp.broadcast_to(regional[:, None, :], (B, H, R))
    return pred_speed.transpose(1, 0, 2), pred_speed_regional


# speedup vs baseline: 1.0398x; 1.0395x over previous
import functools

import jax
import jax.numpy as jnp
from jax import lax
from jax.experimental import pallas as pl
from jax.experimental.pallas import tpu as pltpu
from jax.experimental.pallas import tpu_sc as plsc

B, T, N, F = 16, 20, 50000, 2
H = 10
R = 64
RP = 128
NB = 4096
NSTEPS = (N + NB - 1) // NB
NPAD = NSTEPS * NB

_SC = plsc.get_sparse_core_info()
NC, NS, LANES = _SC.num_cores, _SC.num_subcores, _SC.num_lanes
NW = NC * NS
CH = NPAD // NW
assert CH * NW == NPAD and CH % 8 == 0 and NPAD % NB == 0


def _tc_kernel(x_ref, out_pred_ref, out_flat_ref):
    i = pl.program_id(0)

    wr = jax.lax.broadcasted_iota(jnp.int32, (B, B * T), 0)
    wc = jax.lax.broadcasted_iota(jnp.int32, (B, B * T), 1)
    w = jnp.where(wc // T == wr, 1.0 / T, 0.0)
    x = x_ref[...].reshape(B * T, NB)
    pred = jax.lax.dot_general(
        w, x, (((1,), (0,)), ((), ())),
        preferred_element_type=jnp.float32)

    node_idx = i * NB + jax.lax.broadcasted_iota(jnp.int32, (1, NB), 1)
    pred = jnp.where(node_idx < N, pred, 0.0)

    out_pred_ref[...] = jnp.broadcast_to(pred[None, :, :], (H, B, NB))
    out_flat_ref[...] = pred


_sc_mesh = plsc.VectorSubcoreMesh(core_axis_name="c", subcore_axis_name="s")


@functools.partial(
    pl.kernel,
    mesh=_sc_mesh,
    compiler_params=pltpu.CompilerParams(needs_layout_passes=False),
    out_type=[
        jax.ShapeDtypeStruct((NW, B + 1, R), jnp.float32),
    ],
    scratch_types=[
        pltpu.VMEM((CH,), jnp.int32),
        pltpu.VMEM((B, CH), jnp.float32),
        pltpu.VMEM((RP,), jnp.float32),
        pltpu.VMEM((B + 1, R), jnp.float32),
        pltpu.VMEM((RP,), jnp.float32),
    ],
)
def _sc_segment(pred_hbm, cid_hbm, sums_hbm,
                cid_v, val_v, bins_v, sums_v, cnt_v):
    c = lax.axis_index("c")
    s = lax.axis_index("s")
    wid = s * NC + c
    base = wid * CH
    zeros16 = jnp.zeros((LANES,), jnp.float32)
    ones16 = jnp.ones((LANES,), jnp.float32)

    pltpu.sync_copy(cid_hbm.at[pl.ds(base, CH)], cid_v)
    pltpu.sync_copy(pred_hbm.at[:, pl.ds(base, CH)], val_v)

    for k in range(RP // LANES):
        cnt_v[pl.ds(k * LANES, LANES)] = zeros16

    def _cnt_body(j, carry):
        idx = cid_v[pl.ds(j * LANES, LANES)]
        plsc.addupdate_scatter(cnt_v, [idx], ones16)
        return carry

    lax.fori_loop(0, CH // LANES, _cnt_body, 0)

    for b in range(B):
        for k in range(RP // LANES):
            bins_v[pl.ds(k * LANES, LANES)] = zeros16

        def _sum_body(j, carry):
            idx = cid_v[pl.ds(j * LANES, LANES)]
            vals = val_v[b, pl.ds(j * LANES, LANES)]
            plsc.addupdate_scatter(bins_v, [idx], vals)
            return carry

        lax.fori_loop(0, CH // LANES, _sum_body, 0)
        for k in range(R // LANES):
            sums_v[b, pl.ds(k * LANES, LANES)] = bins_v[pl.ds(k * LANES, LANES)]

    for k in range(R // LANES):
        sums_v[B, pl.ds(k * LANES, LANES)] = cnt_v[pl.ds(k * LANES, LANES)]

    pltpu.sync_copy(sums_v, sums_hbm.at[wid])


def kernel(speed_seq, cluster_id):
    x0 = speed_seq[:, :, :, 0]
    cid = cluster_id.astype(jnp.int32)
    cid_pad = jnp.concatenate([cid, jnp.full((NPAD - N,), R, dtype=jnp.int32)])

    pred_speed, pred_flat = pl.pallas_call(
        _tc_kernel,
        grid=(NSTEPS,),
        in_specs=[pl.BlockSpec((B, T, NB), lambda i: (0, 0, i))],
        out_specs=[
            pl.BlockSpec((H, B, NB), lambda i: (0, 0, i)),
            pl.BlockSpec((B, NB), lambda i: (0, i)),
        ],
        out_shape=[
            jax.ShapeDtypeStruct((H, B, N), jnp.float32),
            jax.ShapeDtypeStruct((B, NPAD), jnp.float32),
        ],
    )(x0)

    (partials,) = _sc_segment(pred_flat, cid_pad)
    combined = partials.sum(axis=0)
    regional = combined[:B] / combined[B][None, :]
    pred_speed_regional = jnp.broadcast_to(regional[:, None, :], (B, H, R))
    return pred_speed.transpose(1, 0, 2), pred_speed_regional
